# speedup vs baseline: 4.2366x; 4.2366x over previous
import functools

import jax
import jax.numpy as jnp
from jax import lax
from jax.experimental import pallas as pl
from jax.experimental.pallas import tpu as pltpu

N = 256
FEAT = 128
LATENT = 256
NF = 5
BI = 8
GRID = N // BI

_DN = (((1,), (1,)), ((), ()))


def _tc_body(embs_ref, W_ref, b_ref, o0, o1, o2, o3, o4, p1_ref, p2_ref):
    ib = pl.program_id(0)

    @pl.when(ib == 0)
    def _():
        for f in range(NF):
            E = embs_ref[f]
            W = W_ref[f]
            p1_ref[f] = lax.dot_general(E, W[:, :FEAT], _DN,
                                        preferred_element_type=jnp.float32)
            p2_ref[f] = (lax.dot_general(E, W[:, FEAT:2 * FEAT], _DN,
                                         preferred_element_type=jnp.float32)
                         + b_ref[f])

    start = ib * BI
    outs = (o0, o1, o2, o3, o4)
    for f in range(NF):
        E = embs_ref[f]
        e_blk = embs_ref[f, pl.ds(start, BI), :]
        R = e_blk[:, None, :] * E[None, :, :]
        M = lax.dot_general(R, W_ref[f][:, 2 * FEAT:],
                            (((2,), (1,)), ((), ())),
                            preferred_element_type=jnp.float32)
        p1_blk = p1_ref[f, pl.ds(start, BI), :]
        out3 = jnp.maximum(M + p1_blk[:, None, :] + p2_ref[f][None, :, :], 0.0)
        outs[f][...] = out3.reshape(BI * N, LATENT)


@functools.partial(jax.jit, static_argnames=("interpret",))
def _encode(embs, W, b, interpret=False):
    return pl.pallas_call(
        _tc_body,
        grid=(GRID,),
        in_specs=[
            pl.BlockSpec((NF, N, FEAT), lambda i: (0, 0, 0)),
            pl.BlockSpec((NF, LATENT, 3 * FEAT), lambda i: (0, 0, 0)),
            pl.BlockSpec((NF, 1, LATENT), lambda i: (0, 0, 0)),
        ],
        out_specs=[pl.BlockSpec((BI * N, LATENT), lambda i: (i, 0))] * NF,
        out_shape=[jax.ShapeDtypeStruct((N * N, LATENT), jnp.float32)] * NF,
        scratch_shapes=[pltpu.VMEM((NF, N, LATENT), jnp.float32)] * 2,
        interpret=interpret,
    )(embs, W, b)


def kernel(typ_ids, typ_table, typ_W, typ_b, pol_ids, pol_table, pol_W, pol_b,
           mod_ids, mod_table, mod_W, mod_b, gen_ids, gen_table, gen_W, gen_b,
           ten_ids, ten_table, ten_W, ten_b):
    ids = (typ_ids, pol_ids, mod_ids, gen_ids, ten_ids)
    tables = (typ_table, pol_table, mod_table, gen_table, ten_table)
    embs = jnp.stack([jnp.take(t, i, axis=0) for t, i in zip(tables, ids)])
    W = jnp.stack((typ_W, pol_W, mod_W, gen_W, ten_W))
    b = jnp.stack((typ_b, pol_b, mod_b, gen_b, ten_b)).reshape(NF, 1, LATENT)
    return tuple(_encode(embs, W, b))
